# baseline (speedup 1.0000x reference)
import functools

import jax
import jax.numpy as jnp
import numpy as np
from jax import lax
from jax.experimental import pallas as pl
from jax.experimental.pallas import tpu as pltpu
from jax.experimental.pallas import tpu_sc as plsc

N = 10000
E = 320000
D = 128
H = 128
OUT = 64
G = 64
L = 3

NC = 2
NS = 16
NW = NC * NS
CH = 128
EPT = 10240
EPAD = EPT * NW
CHUNKS = EPT // CH
SEG = 16
NSEG = CHUNKS // SEG
K0 = 128
K1 = 32
NROWS = 10240
RPT = NROWS // NS
NDUMMY = NROWS - N

RSQ = float(1.0 / np.sqrt(1.0 + 1e-5))

RB = 400
NBLK = N // RB


def _sc_segment_sum(h, src, dst, zrows):
    mesh = plsc.VectorSubcoreMesh(core_axis_name="c", subcore_axis_name="s")

    @functools.partial(
        pl.kernel,
        mesh=mesh,
        out_type=jax.ShapeDtypeStruct((2 * N, D), jnp.float32),
        scratch_types=[
            pltpu.VMEM((SEG, CH), jnp.int32),
            pltpu.VMEM((SEG, CH), jnp.int32),
            pltpu.VMEM((SEG, CH), jnp.int32),
            pltpu.VMEM((SEG, CH), jnp.int32),
            pltpu.VMEM((CH, D), jnp.float32),
            pltpu.VMEM((CH, D), jnp.float32),
            pltpu.VMEM_SHARED((NROWS, D), jnp.float32),
            pltpu.SemaphoreType.DMA,
            pltpu.SemaphoreType.DMA,
            pltpu.SemaphoreType.DMA,
            pltpu.SemaphoreType.DMA,
        ],
    )
    def k(h_hbm, src_hbm, dst_hbm, z_hbm, out_hbm,
          sA, dA, sB, dB, rows0, rows1, agg, sem0, sem1, semia, semib):
        c = lax.axis_index("c")
        s = lax.axis_index("s")
        cb = (c * NS + s) * CHUNKS

        pltpu.sync_copy(z_hbm, rows0)
        for r in range(RPT // CH):
            pltpu.sync_copy(rows0, agg.at[pl.ds(s * RPT + r * CH, CH)])
        plsc.subcore_barrier()

        def seg_run(s_ref, d_ref):
            g0 = pltpu.async_copy(h_hbm.at[s_ref.at[0]], rows0, sem0)

            def body(jj, carry):
                j = 2 * jj
                g1 = pltpu.async_copy(h_hbm.at[s_ref.at[j + 1]], rows1, sem1)
                g0.wait()
                pltpu.sync_copy(rows0, agg.at[d_ref.at[j]], add=True)
                pltpu.async_copy(h_hbm.at[s_ref.at[j + 2]], rows0, sem0)
                g1.wait()
                pltpu.sync_copy(rows1, agg.at[d_ref.at[j + 1]], add=True)
                return carry

            lax.fori_loop(0, SEG // 2 - 1, body, 0)
            g1 = pltpu.async_copy(h_hbm.at[s_ref.at[SEG - 1]], rows1, sem1)
            g0.wait()
            pltpu.sync_copy(rows0, agg.at[d_ref.at[SEG - 2]], add=True)
            g1.wait()
            pltpu.sync_copy(rows1, agg.at[d_ref.at[SEG - 1]], add=True)

        def run_all(base, nseg):
            pltpu.sync_copy(src_hbm.at[pl.ds(base, SEG)], sA)
            pltpu.sync_copy(dst_hbm.at[pl.ds(base, SEG)], dA)
            pend = []
            for seg in range(nseg):
                cur_s, cur_d = (sA, dA) if seg % 2 == 0 else (sB, dB)
                for pnd in pend:
                    pnd.wait()
                if seg + 1 < nseg:
                    nxt_s, nxt_d = (sB, dB) if seg % 2 == 0 else (sA, dA)
                    off = base + (seg + 1) * SEG
                    sem = semib if seg % 2 == 0 else semia
                    pend = [
                        pltpu.async_copy(src_hbm.at[pl.ds(off, SEG)], nxt_s, sem),
                        pltpu.async_copy(dst_hbm.at[pl.ds(off, SEG)], nxt_d, sem),
                    ]
                else:
                    pend = []
                seg_run(cur_s, cur_d)

        @pl.when(c == 0)
        def _():
            run_all(NS * K1 + s * K0, K0 // SEG)

        @pl.when(c == 1)
        def _():
            run_all(s * K1, K1 // SEG)

        plsc.subcore_barrier()

        @pl.when(s < NS - 1)
        def _():
            pltpu.sync_copy(
                agg.at[pl.ds(s * RPT, RPT)],
                out_hbm.at[pl.ds(c * N + s * RPT, RPT)],
            )

        @pl.when(s == NS - 1)
        def _():
            pltpu.sync_copy(
                agg.at[pl.ds((NS - 1) * RPT, N - (NS - 1) * RPT)],
                out_hbm.at[pl.ds(c * N + (NS - 1) * RPT, N - (NS - 1) * RPT)],
            )

    return k(h, src, dst, zrows)


def _tc_layer_body(h_ref, a0_ref, a1_ref, w1_ref, b1_ref, g1_ref, be1_ref,
                   w2_ref, b2_ref, g2_ref, be2_ref, mul_ref, out_ref):
    z = mul_ref[0, 0] * h_ref[...] + a0_ref[...] + a1_ref[...]
    z = jnp.dot(z, w1_ref[...], preferred_element_type=jnp.float32)
    z = (z + b1_ref[...]) * (g1_ref[...] * RSQ) + be1_ref[...]
    z = jnp.maximum(z, 0.0)
    z = jnp.dot(z, w2_ref[...], preferred_element_type=jnp.float32)
    z = (z + b2_ref[...]) * (g2_ref[...] * RSQ) + be2_ref[...]
    out_ref[...] = jnp.maximum(z, 0.0)


def _row_spec():
    return pl.BlockSpec((RB, D), lambda i: (i, 0))


def _agg1_spec():
    return pl.BlockSpec((RB, D), lambda i: (NBLK + i, 0))


def _full_spec(shape):
    nd = len(shape)
    return pl.BlockSpec(shape, lambda i: (0,) * nd)


def _tc_layer(h, parts, w1, b1, g1, be1, w2, b2, g2, be2, mul):
    return pl.pallas_call(
        _tc_layer_body,
        grid=(NBLK,),
        in_specs=[
            _row_spec(), _row_spec(), _agg1_spec(),
            _full_spec((D, H)), _full_spec((1, H)), _full_spec((1, H)),
            _full_spec((1, H)),
            _full_spec((H, H)), _full_spec((1, H)), _full_spec((1, H)),
            _full_spec((1, H)),
            pl.BlockSpec(memory_space=pltpu.SMEM),
        ],
        out_specs=_row_spec(),
        out_shape=jax.ShapeDtypeStruct((N, D), jnp.float32),
    )(h, parts, parts, w1, b1, g1, be1, w2, b2, g2, be2, mul)


def _tc_final_body(h_ref, a0_ref, a1_ref, w1_ref, b1_ref, g1_ref, be1_ref,
                   w2_ref, b2_ref, g2_ref, be2_ref, mul_ref, batch_ref,
                   cw1_ref, cb1_ref, cw2_ref, cb2_ref, out_ref, pool_acc):
    i = pl.program_id(0)

    z = mul_ref[0, 0] * h_ref[...] + a0_ref[...] + a1_ref[...]
    z = jnp.dot(z, w1_ref[...], preferred_element_type=jnp.float32)
    z = (z + b1_ref[...]) * (g1_ref[...] * RSQ) + be1_ref[...]
    z = jnp.maximum(z, 0.0)
    z = jnp.dot(z, w2_ref[...], preferred_element_type=jnp.float32)
    z = (z + b2_ref[...]) * (g2_ref[...] * RSQ) + be2_ref[...]
    z = jnp.maximum(z, 0.0)

    b = batch_ref[0]
    onehot = (b == lax.broadcasted_iota(jnp.int32, (G, RB), 0)).astype(jnp.float32)
    part = jnp.dot(onehot, z, preferred_element_type=jnp.float32)

    @pl.when(i == 0)
    def _():
        pool_acc[...] = part

    @pl.when(i > 0)
    def _():
        pool_acc[...] += part

    @pl.when(i == NBLK - 1)
    def _():
        p = jnp.dot(pool_acc[...], cw1_ref[...], preferred_element_type=jnp.float32)
        p = jnp.maximum(p + cb1_ref[...], 0.0)
        out_ref[...] = (
            jnp.dot(p, cw2_ref[...], preferred_element_type=jnp.float32)
            + cb2_ref[...]
        )


def _tc_final(h, parts, w1, b1, g1, be1, w2, b2, g2, be2, mul, batch_r,
              cw1, cb1, cw2, cb2):
    return pl.pallas_call(
        _tc_final_body,
        grid=(NBLK,),
        in_specs=[
            _row_spec(), _row_spec(), _agg1_spec(),
            _full_spec((D, H)), _full_spec((1, H)), _full_spec((1, H)),
            _full_spec((1, H)),
            _full_spec((H, H)), _full_spec((1, H)), _full_spec((1, H)),
            _full_spec((1, H)),
            pl.BlockSpec(memory_space=pltpu.SMEM),
            pl.BlockSpec((1, 1, RB), lambda i: (i, 0, 0)),
            _full_spec((H, H)), _full_spec((1, H)),
            _full_spec((H, OUT)), _full_spec((1, OUT)),
        ],
        out_specs=pl.BlockSpec((G, OUT), lambda i: (0, 0)),
        out_shape=jax.ShapeDtypeStruct((G, OUT), jnp.float32),
        scratch_shapes=[pltpu.VMEM((G, D), jnp.float32)],
    )(h, parts, parts, w1, b1, g1, be1, w2, b2, g2, be2, mul, batch_r,
      cw1, cb1, cw2, cb2)


@jax.jit
def kernel(x, ei, batch, eps, w1, b1, g1, be1, w2, b2, g2, be2, cw1, cb1, cw2, cb2):
    pad = EPAD - E
    src = jnp.concatenate([ei[0], jnp.zeros((pad,), jnp.int32)])
    src = src.reshape(NW * CHUNKS, CH)
    pad_dst = N + (jnp.arange(pad, dtype=jnp.int32) % NDUMMY)
    dst = jnp.concatenate([ei[1], pad_dst])
    dst = dst.reshape(NW * CHUNKS, CH)
    zrows = jnp.zeros((CH, D), jnp.float32)
    batch_r = batch.reshape(NBLK, 1, RB)
    mul = (1.0 + eps).reshape(L, 1, 1)

    h = x
    for i in range(L - 1):
        parts = _sc_segment_sum(h, src, dst, zrows)
        h = _tc_layer(h, parts,
                      w1[i], b1[i].reshape(1, H), g1[i].reshape(1, H),
                      be1[i].reshape(1, H),
                      w2[i], b2[i].reshape(1, H), g2[i].reshape(1, H),
                      be2[i].reshape(1, H), mul[i])

    i = L - 1
    parts = _sc_segment_sum(h, src, dst, zrows)
    return _tc_final(h, parts,
                     w1[i], b1[i].reshape(1, H), g1[i].reshape(1, H),
                     be1[i].reshape(1, H),
                     w2[i], b2[i].reshape(1, H), g2[i].reshape(1, H),
                     be2[i].reshape(1, H), mul[i], batch_r,
                     cw1, cb1.reshape(1, H), cw2, cb2.reshape(1, OUT))


# speedup vs baseline: 3.3789x; 3.3789x over previous
import functools

import jax
import jax.numpy as jnp
import numpy as np
from jax import lax
from jax.experimental import pallas as pl
from jax.experimental.pallas import tpu as pltpu
from jax.experimental.pallas import tpu_sc as plsc

N = 10000
E = 320000
D = 128
H = 128
OUT = 64
G = 64
L = 3

NC = 2
NS = 16
NW = NC * NS
CH = 128
EPT = 10240
EPAD = EPT * NW
CHUNKS = EPT // CH
SEG = 16
NSEG = CHUNKS // SEG
K0 = 128
K1 = 32
NROWS = 10240
RPT = NROWS // NS
NDUMMY = NROWS - N

RSQ = float(1.0 / np.sqrt(1.0 + 1e-5))

RB = 400
NBLK = N // RB


def _sc_segment_sum(h, src, dst, zrows):
    mesh = plsc.VectorSubcoreMesh(core_axis_name="c", subcore_axis_name="s")

    @functools.partial(
        pl.kernel,
        mesh=mesh,
        out_type=jax.ShapeDtypeStruct((2 * N, D), jnp.float32),
        scratch_types=[
            pltpu.VMEM((SEG, CH), jnp.int32),
            pltpu.VMEM((SEG, CH), jnp.int32),
            pltpu.VMEM((SEG, CH), jnp.int32),
            pltpu.VMEM((SEG, CH), jnp.int32),
            pltpu.VMEM((CH, D), jnp.float32),
            pltpu.VMEM((CH, D), jnp.float32),
            pltpu.VMEM_SHARED((NROWS, D), jnp.float32),
            pltpu.SemaphoreType.DMA,
            pltpu.SemaphoreType.DMA,
            pltpu.SemaphoreType.DMA,
            pltpu.SemaphoreType.DMA,
        ],
    )
    def k(h_hbm, src_hbm, dst_hbm, z_hbm, out_hbm,
          sA, dA, sB, dB, rows0, rows1, agg, sem0, sem1, semia, semib):
        c = lax.axis_index("c")
        s = lax.axis_index("s")
        cb = (c * NS + s) * CHUNKS

        pltpu.sync_copy(z_hbm, rows0)
        for r in range(RPT // CH):
            pltpu.sync_copy(rows0, agg.at[pl.ds(s * RPT + r * CH, CH)])
        plsc.subcore_barrier()

        def seg_run(s_ref, d_ref):
            g0 = pltpu.async_copy(h_hbm.at[s_ref.at[0]], rows0, sem0)

            def body(jj, carry):
                j = 2 * jj
                g1 = pltpu.async_copy(h_hbm.at[s_ref.at[j + 1]], rows1, sem1)
                g0.wait()
                pltpu.sync_copy(rows0, agg.at[d_ref.at[j]], add=True)
                pltpu.async_copy(h_hbm.at[s_ref.at[j + 2]], rows0, sem0)
                g1.wait()
                pltpu.sync_copy(rows1, agg.at[d_ref.at[j + 1]], add=True)
                return carry

            lax.fori_loop(0, SEG // 2 - 1, body, 0)
            g1 = pltpu.async_copy(h_hbm.at[s_ref.at[SEG - 1]], rows1, sem1)
            g0.wait()
            pltpu.sync_copy(rows0, agg.at[d_ref.at[SEG - 2]], add=True)
            g1.wait()
            pltpu.sync_copy(rows1, agg.at[d_ref.at[SEG - 1]], add=True)

        def run_all(base, nseg):
            pltpu.sync_copy(src_hbm.at[pl.ds(base, SEG)], sA)
            pltpu.sync_copy(dst_hbm.at[pl.ds(base, SEG)], dA)
            pend = []
            for seg in range(nseg):
                cur_s, cur_d = (sA, dA) if seg % 2 == 0 else (sB, dB)
                for pnd in pend:
                    pnd.wait()
                if seg + 1 < nseg:
                    nxt_s, nxt_d = (sB, dB) if seg % 2 == 0 else (sA, dA)
                    off = base + (seg + 1) * SEG
                    sem = semib if seg % 2 == 0 else semia
                    pend = [
                        pltpu.async_copy(src_hbm.at[pl.ds(off, SEG)], nxt_s, sem),
                        pltpu.async_copy(dst_hbm.at[pl.ds(off, SEG)], nxt_d, sem),
                    ]
                else:
                    pend = []
                seg_run(cur_s, cur_d)

        run_all(cb, NSEG)
        plsc.subcore_barrier()

        @pl.when(s < NS - 1)
        def _():
            pltpu.sync_copy(
                agg.at[pl.ds(s * RPT, RPT)],
                out_hbm.at[pl.ds(c * N + s * RPT, RPT)],
            )

        @pl.when(s == NS - 1)
        def _():
            pltpu.sync_copy(
                agg.at[pl.ds((NS - 1) * RPT, N - (NS - 1) * RPT)],
                out_hbm.at[pl.ds(c * N + (NS - 1) * RPT, N - (NS - 1) * RPT)],
            )

    return k(h, src, dst, zrows)


def _tc_layer_body(h_ref, a0_ref, a1_ref, w1_ref, b1_ref, g1_ref, be1_ref,
                   w2_ref, b2_ref, g2_ref, be2_ref, mul_ref, out_ref):
    z = mul_ref[0, 0] * h_ref[...] + a0_ref[...] + a1_ref[...]
    z = jnp.dot(z, w1_ref[...], preferred_element_type=jnp.float32)
    z = (z + b1_ref[...]) * (g1_ref[...] * RSQ) + be1_ref[...]
    z = jnp.maximum(z, 0.0)
    z = jnp.dot(z, w2_ref[...], preferred_element_type=jnp.float32)
    z = (z + b2_ref[...]) * (g2_ref[...] * RSQ) + be2_ref[...]
    out_ref[...] = jnp.maximum(z, 0.0)


def _row_spec():
    return pl.BlockSpec((RB, D), lambda i: (i, 0))


def _agg1_spec():
    return pl.BlockSpec((RB, D), lambda i: (NBLK + i, 0))


def _full_spec(shape):
    nd = len(shape)
    return pl.BlockSpec(shape, lambda i: (0,) * nd)


def _tc_layer(h, parts, w1, b1, g1, be1, w2, b2, g2, be2, mul):
    return pl.pallas_call(
        _tc_layer_body,
        grid=(NBLK,),
        in_specs=[
            _row_spec(), _row_spec(), _agg1_spec(),
            _full_spec((D, H)), _full_spec((1, H)), _full_spec((1, H)),
            _full_spec((1, H)),
            _full_spec((H, H)), _full_spec((1, H)), _full_spec((1, H)),
            _full_spec((1, H)),
            pl.BlockSpec(memory_space=pltpu.SMEM),
        ],
        out_specs=_row_spec(),
        out_shape=jax.ShapeDtypeStruct((N, D), jnp.float32),
    )(h, parts, parts, w1, b1, g1, be1, w2, b2, g2, be2, mul)


def _tc_final_body(h_ref, a0_ref, a1_ref, w1_ref, b1_ref, g1_ref, be1_ref,
                   w2_ref, b2_ref, g2_ref, be2_ref, mul_ref, batch_ref,
                   cw1_ref, cb1_ref, cw2_ref, cb2_ref, out_ref, pool_acc):
    i = pl.program_id(0)

    z = mul_ref[0, 0] * h_ref[...] + a0_ref[...] + a1_ref[...]
    z = jnp.dot(z, w1_ref[...], preferred_element_type=jnp.float32)
    z = (z + b1_ref[...]) * (g1_ref[...] * RSQ) + be1_ref[...]
    z = jnp.maximum(z, 0.0)
    z = jnp.dot(z, w2_ref[...], preferred_element_type=jnp.float32)
    z = (z + b2_ref[...]) * (g2_ref[...] * RSQ) + be2_ref[...]
    z = jnp.maximum(z, 0.0)

    b = batch_ref[0]
    onehot = (b == lax.broadcasted_iota(jnp.int32, (G, RB), 0)).astype(jnp.float32)
    part = jnp.dot(onehot, z, preferred_element_type=jnp.float32)

    @pl.when(i == 0)
    def _():
        pool_acc[...] = part

    @pl.when(i > 0)
    def _():
        pool_acc[...] += part

    @pl.when(i == NBLK - 1)
    def _():
        p = jnp.dot(pool_acc[...], cw1_ref[...], preferred_element_type=jnp.float32)
        p = jnp.maximum(p + cb1_ref[...], 0.0)
        out_ref[...] = (
            jnp.dot(p, cw2_ref[...], preferred_element_type=jnp.float32)
            + cb2_ref[...]
        )


def _tc_final(h, parts, w1, b1, g1, be1, w2, b2, g2, be2, mul, batch_r,
              cw1, cb1, cw2, cb2):
    return pl.pallas_call(
        _tc_final_body,
        grid=(NBLK,),
        in_specs=[
            _row_spec(), _row_spec(), _agg1_spec(),
            _full_spec((D, H)), _full_spec((1, H)), _full_spec((1, H)),
            _full_spec((1, H)),
            _full_spec((H, H)), _full_spec((1, H)), _full_spec((1, H)),
            _full_spec((1, H)),
            pl.BlockSpec(memory_space=pltpu.SMEM),
            pl.BlockSpec((1, 1, RB), lambda i: (i, 0, 0)),
            _full_spec((H, H)), _full_spec((1, H)),
            _full_spec((H, OUT)), _full_spec((1, OUT)),
        ],
        out_specs=pl.BlockSpec((G, OUT), lambda i: (0, 0)),
        out_shape=jax.ShapeDtypeStruct((G, OUT), jnp.float32),
        scratch_shapes=[pltpu.VMEM((G, D), jnp.float32)],
    )(h, parts, parts, w1, b1, g1, be1, w2, b2, g2, be2, mul, batch_r,
      cw1, cb1, cw2, cb2)


@jax.jit
def kernel(x, ei, batch, eps, w1, b1, g1, be1, w2, b2, g2, be2, cw1, cb1, cw2, cb2):
    pad = EPAD - E
    pad_src = jnp.arange(pad, dtype=jnp.int32) % N
    src = jnp.concatenate([ei[0], pad_src])
    src = src.reshape(NW * CHUNKS, CH)
    pad_dst = N + (jnp.arange(pad, dtype=jnp.int32) % NDUMMY)
    dst = jnp.concatenate([ei[1], pad_dst])
    dst = dst.reshape(NW * CHUNKS, CH)
    zrows = jnp.zeros((CH, D), jnp.float32)
    batch_r = batch.reshape(NBLK, 1, RB)
    mul = (1.0 + eps).reshape(L, 1, 1)

    h = x
    for i in range(L - 1):
        parts = _sc_segment_sum(h, src, dst, zrows)
        h = _tc_layer(h, parts,
                      w1[i], b1[i].reshape(1, H), g1[i].reshape(1, H),
                      be1[i].reshape(1, H),
                      w2[i], b2[i].reshape(1, H), g2[i].reshape(1, H),
                      be2[i].reshape(1, H), mul[i])

    i = L - 1
    parts = _sc_segment_sum(h, src, dst, zrows)
    return _tc_final(h, parts,
                     w1[i], b1[i].reshape(1, H), g1[i].reshape(1, H),
                     be1[i].reshape(1, H),
                     w2[i], b2[i].reshape(1, H), g2[i].reshape(1, H),
                     be2[i].reshape(1, H), mul[i], batch_r,
                     cw1, cb1.reshape(1, H), cw2, cb2.reshape(1, OUT))


# speedup vs baseline: 3.7025x; 1.0958x over previous
import functools

import jax
import jax.numpy as jnp
import numpy as np
from jax import lax
from jax.experimental import pallas as pl
from jax.experimental.pallas import tpu as pltpu
from jax.experimental.pallas import tpu_sc as plsc

N = 10000
E = 320000
D = 128
H = 128
OUT = 64
G = 64
L = 3

NC = 2
NS = 16
NW = NC * NS
CH = 128
EPT = 10240
EPAD = EPT * NW
CHUNKS = EPT // CH
SEG = 16
NSEG = CHUNKS // SEG
K0 = 128
K1 = 32
NROWS = 10240
RPT = NROWS // NS
NDUMMY = NROWS - N

RSQ = float(1.0 / np.sqrt(1.0 + 1e-5))

RB = 2000
NBLK = N // RB


def _sc_segment_sum(h, src, dst, zrows):
    mesh = plsc.VectorSubcoreMesh(core_axis_name="c", subcore_axis_name="s")

    @functools.partial(
        pl.kernel,
        mesh=mesh,
        out_type=jax.ShapeDtypeStruct((2 * N, D), jnp.float32),
        scratch_types=[
            pltpu.VMEM((SEG, CH), jnp.int32),
            pltpu.VMEM((SEG, CH), jnp.int32),
            pltpu.VMEM((SEG, CH), jnp.int32),
            pltpu.VMEM((SEG, CH), jnp.int32),
            pltpu.VMEM((CH, D), jnp.float32),
            pltpu.VMEM((CH, D), jnp.float32),
            pltpu.VMEM_SHARED((NROWS, D), jnp.float32),
            pltpu.SemaphoreType.DMA,
            pltpu.SemaphoreType.DMA,
            pltpu.SemaphoreType.DMA,
            pltpu.SemaphoreType.DMA,
        ],
    )
    def k(h_hbm, src_hbm, dst_hbm, z_hbm, out_hbm,
          sA, dA, sB, dB, rows0, rows1, agg, sem0, sem1, semia, semib):
        c = lax.axis_index("c")
        s = lax.axis_index("s")
        cb = (c * NS + s) * CHUNKS

        pend = {
            0: [pltpu.async_copy(src_hbm.at[pl.ds(cb, SEG)], sA, semia),
                pltpu.async_copy(dst_hbm.at[pl.ds(cb, SEG)], dA, semia)],
            1: [pltpu.async_copy(src_hbm.at[pl.ds(cb + SEG, SEG)], sB, semib),
                pltpu.async_copy(dst_hbm.at[pl.ds(cb + SEG, SEG)], dB, semib)],
        }

        pltpu.sync_copy(z_hbm, rows0)
        for r in range(RPT // CH):
            pltpu.sync_copy(rows0, agg.at[pl.ds(s * RPT + r * CH, CH)])
        plsc.subcore_barrier()

        def seg_run(s_ref, d_ref):
            g0 = pltpu.async_copy(h_hbm.at[s_ref.at[0]], rows0, sem0)

            def body(jj, carry):
                j = 2 * jj
                g1 = pltpu.async_copy(h_hbm.at[s_ref.at[j + 1]], rows1, sem1)
                g0.wait()
                pltpu.sync_copy(rows0, agg.at[d_ref.at[j]], add=True)
                pltpu.async_copy(h_hbm.at[s_ref.at[j + 2]], rows0, sem0)
                g1.wait()
                pltpu.sync_copy(rows1, agg.at[d_ref.at[j + 1]], add=True)
                return carry

            lax.fori_loop(0, SEG // 2 - 1, body, 0)
            g1 = pltpu.async_copy(h_hbm.at[s_ref.at[SEG - 1]], rows1, sem1)
            g0.wait()
            pltpu.sync_copy(rows0, agg.at[d_ref.at[SEG - 2]], add=True)
            g1.wait()
            pltpu.sync_copy(rows1, agg.at[d_ref.at[SEG - 1]], add=True)

        for seg in range(NSEG):
            cur_s, cur_d = (sA, dA) if seg % 2 == 0 else (sB, dB)
            sem = semia if seg % 2 == 0 else semib
            for pnd in pend.pop(seg, []):
                pnd.wait()
            seg_run(cur_s, cur_d)
            if seg + 2 < NSEG:
                off = cb + (seg + 2) * SEG
                pend[seg + 2] = [
                    pltpu.async_copy(src_hbm.at[pl.ds(off, SEG)], cur_s, sem),
                    pltpu.async_copy(dst_hbm.at[pl.ds(off, SEG)], cur_d, sem),
                ]
        plsc.subcore_barrier()

        @pl.when(s < NS - 1)
        def _():
            pltpu.sync_copy(
                agg.at[pl.ds(s * RPT, RPT)],
                out_hbm.at[pl.ds(c * N + s * RPT, RPT)],
            )

        @pl.when(s == NS - 1)
        def _():
            pltpu.sync_copy(
                agg.at[pl.ds((NS - 1) * RPT, N - (NS - 1) * RPT)],
                out_hbm.at[pl.ds(c * N + (NS - 1) * RPT, N - (NS - 1) * RPT)],
            )

    return k(h, src, dst, zrows)


def _tc_layer_body(h_ref, a0_ref, a1_ref, w1_ref, b1_ref, g1_ref, be1_ref,
                   w2_ref, b2_ref, g2_ref, be2_ref, mul_ref, out_ref):
    z = mul_ref[0, 0] * h_ref[...] + a0_ref[...] + a1_ref[...]
    z = jnp.dot(z, w1_ref[...], preferred_element_type=jnp.float32)
    z = (z + b1_ref[...]) * (g1_ref[...] * RSQ) + be1_ref[...]
    z = jnp.maximum(z, 0.0)
    z = jnp.dot(z, w2_ref[...], preferred_element_type=jnp.float32)
    z = (z + b2_ref[...]) * (g2_ref[...] * RSQ) + be2_ref[...]
    out_ref[...] = jnp.maximum(z, 0.0)


def _row_spec():
    return pl.BlockSpec((RB, D), lambda i: (i, 0))


def _agg1_spec():
    return pl.BlockSpec((RB, D), lambda i: (NBLK + i, 0))


def _full_spec(shape):
    nd = len(shape)
    return pl.BlockSpec(shape, lambda i: (0,) * nd)


def _tc_layer(h, parts, w1, b1, g1, be1, w2, b2, g2, be2, mul):
    return pl.pallas_call(
        _tc_layer_body,
        grid=(NBLK,),
        in_specs=[
            _row_spec(), _row_spec(), _agg1_spec(),
            _full_spec((D, H)), _full_spec((1, H)), _full_spec((1, H)),
            _full_spec((1, H)),
            _full_spec((H, H)), _full_spec((1, H)), _full_spec((1, H)),
            _full_spec((1, H)),
            pl.BlockSpec(memory_space=pltpu.SMEM),
        ],
        out_specs=_row_spec(),
        out_shape=jax.ShapeDtypeStruct((N, D), jnp.float32),
    )(h, parts, parts, w1, b1, g1, be1, w2, b2, g2, be2, mul)


def _tc_final_body(h_ref, a0_ref, a1_ref, w1_ref, b1_ref, g1_ref, be1_ref,
                   w2_ref, b2_ref, g2_ref, be2_ref, mul_ref, batch_ref,
                   cw1_ref, cb1_ref, cw2_ref, cb2_ref, out_ref, pool_acc):
    i = pl.program_id(0)

    z = mul_ref[0, 0] * h_ref[...] + a0_ref[...] + a1_ref[...]
    z = jnp.dot(z, w1_ref[...], preferred_element_type=jnp.float32)
    z = (z + b1_ref[...]) * (g1_ref[...] * RSQ) + be1_ref[...]
    z = jnp.maximum(z, 0.0)
    z = jnp.dot(z, w2_ref[...], preferred_element_type=jnp.float32)
    z = (z + b2_ref[...]) * (g2_ref[...] * RSQ) + be2_ref[...]
    z = jnp.maximum(z, 0.0)

    b = batch_ref[0]
    onehot = (b == lax.broadcasted_iota(jnp.int32, (G, RB), 0)).astype(jnp.float32)
    part = jnp.dot(onehot, z, preferred_element_type=jnp.float32)

    @pl.when(i == 0)
    def _():
        pool_acc[...] = part

    @pl.when(i > 0)
    def _():
        pool_acc[...] += part

    @pl.when(i == NBLK - 1)
    def _():
        p = jnp.dot(pool_acc[...], cw1_ref[...], preferred_element_type=jnp.float32)
        p = jnp.maximum(p + cb1_ref[...], 0.0)
        out_ref[...] = (
            jnp.dot(p, cw2_ref[...], preferred_element_type=jnp.float32)
            + cb2_ref[...]
        )


def _tc_final(h, parts, w1, b1, g1, be1, w2, b2, g2, be2, mul, batch_r,
              cw1, cb1, cw2, cb2):
    return pl.pallas_call(
        _tc_final_body,
        grid=(NBLK,),
        in_specs=[
            _row_spec(), _row_spec(), _agg1_spec(),
            _full_spec((D, H)), _full_spec((1, H)), _full_spec((1, H)),
            _full_spec((1, H)),
            _full_spec((H, H)), _full_spec((1, H)), _full_spec((1, H)),
            _full_spec((1, H)),
            pl.BlockSpec(memory_space=pltpu.SMEM),
            pl.BlockSpec((1, 1, RB), lambda i: (i, 0, 0)),
            _full_spec((H, H)), _full_spec((1, H)),
            _full_spec((H, OUT)), _full_spec((1, OUT)),
        ],
        out_specs=pl.BlockSpec((G, OUT), lambda i: (0, 0)),
        out_shape=jax.ShapeDtypeStruct((G, OUT), jnp.float32),
        scratch_shapes=[pltpu.VMEM((G, D), jnp.float32)],
    )(h, parts, parts, w1, b1, g1, be1, w2, b2, g2, be2, mul, batch_r,
      cw1, cb1, cw2, cb2)


@jax.jit
def kernel(x, ei, batch, eps, w1, b1, g1, be1, w2, b2, g2, be2, cw1, cb1, cw2, cb2):
    pad = EPAD - E
    pad_src = jnp.arange(pad, dtype=jnp.int32) % N
    src = jnp.concatenate([ei[0], pad_src])
    src = src.reshape(NW * CHUNKS, CH)
    pad_dst = N + (jnp.arange(pad, dtype=jnp.int32) % NDUMMY)
    dst = jnp.concatenate([ei[1], pad_dst])
    dst = dst.reshape(NW * CHUNKS, CH)
    zrows = jnp.zeros((CH, D), jnp.float32)
    batch_r = batch.reshape(NBLK, 1, RB)
    mul = (1.0 + eps).reshape(L, 1, 1)

    h = x
    for i in range(L - 1):
        parts = _sc_segment_sum(h, src, dst, zrows)
        h = _tc_layer(h, parts,
                      w1[i], b1[i].reshape(1, H), g1[i].reshape(1, H),
                      be1[i].reshape(1, H),
                      w2[i], b2[i].reshape(1, H), g2[i].reshape(1, H),
                      be2[i].reshape(1, H), mul[i])

    i = L - 1
    parts = _sc_segment_sum(h, src, dst, zrows)
    return _tc_final(h, parts,
                     w1[i], b1[i].reshape(1, H), g1[i].reshape(1, H),
                     be1[i].reshape(1, H),
                     w2[i], b2[i].reshape(1, H), g2[i].reshape(1, H),
                     be2[i].reshape(1, H), mul[i], batch_r,
                     cw1, cb1.reshape(1, H), cw2, cb2.reshape(1, OUT))
